# Initial kernel scaffold; baseline (speedup 1.0000x reference)
import jax
import jax.numpy as jnp
from jax.experimental import pallas as pl


def kernel(x, gate_w, gate_b, w1, b1, w2, b2, gamma, beta):
    raise NotImplementedError("write your pallas kernel here")


# speedup vs baseline: 4.4821x; 4.4821x over previous
import functools

import jax
import jax.numpy as jnp
from jax.experimental import pallas as pl
from jax.experimental.pallas import tpu as pltpu


def _moe_body(x_ref, xb_ref, gw_ref, gb_ref, w1_ref, b1_ref, w2_ref,
              b2_ref, gamma_ref, beta_ref, out_ref, c_s, acc_s, *, ne, nhc):
    e = pl.program_id(0)
    hc = pl.program_id(1)
    T = x_ref.shape[0]
    E = gw_ref.shape[1]

    @pl.when((e == 0) & (hc == 0))
    def _gate():
        logits = jnp.dot(x_ref[...], gw_ref[...],
                         preferred_element_type=jnp.float32) + gb_ref[...]
        ids = jax.lax.broadcasted_iota(jnp.int32, (T, E), 1)
        m1 = jnp.max(logits, axis=1, keepdims=True)
        i1 = jnp.min(jnp.where(logits == m1, ids, E), axis=1, keepdims=True)
        masked = jnp.where(ids == i1, -jnp.inf, logits)
        m2 = jnp.max(masked, axis=1, keepdims=True)
        i2 = jnp.min(jnp.where(masked == m2, ids, E), axis=1, keepdims=True)
        p1 = 1.0 / (1.0 + jnp.exp(m2 - m1))
        c_s[...] = jnp.where(ids == i1, p1, 0.0) + \
            jnp.where(ids == i2, 1.0 - p1, 0.0)
        acc_s[...] = jnp.zeros_like(acc_s)

    ids = jax.lax.broadcasted_iota(jnp.int32, (T, E), 1)
    ce = jnp.sum(jnp.where(ids == e, c_s[...], 0.0), axis=1, keepdims=True)

    h = jnp.dot(xb_ref[...], w1_ref[0],
                preferred_element_type=jnp.float32) + b1_ref[0]
    h = 0.5 * h * (1.0 + jax.lax.erf(h * 0.7071067811865476))
    hb = (h * ce).astype(jnp.bfloat16)
    acc_s[...] += jnp.dot(hb, w2_ref[0], preferred_element_type=jnp.float32)

    @pl.when(hc == 0)
    def _bias2():
        acc_s[...] += b2_ref[0] * ce

    @pl.when((e == ne - 1) & (hc == nhc - 1))
    def _finish():
        y = x_ref[...] + acc_s[...]
        mu = jnp.mean(y, axis=1, keepdims=True)
        var = jnp.mean((y - mu) ** 2, axis=1, keepdims=True)
        out_ref[...] = (y - mu) / jnp.sqrt(var + 1e-5) * gamma_ref[...] \
            + beta_ref[...]


def kernel(x, gate_w, gate_b, w1, b1, w2, b2, gamma, beta):
    B, T, D = x.shape
    E = gate_w.shape[1]
    H = w1.shape[2]
    HC = min(512, H)
    nhc = H // HC

    x2 = x.reshape(T, D)
    xb = x2.astype(jnp.bfloat16)
    w1b = w1.astype(jnp.bfloat16)
    w2b = w2.astype(jnp.bfloat16)
    b1r = b1.reshape(E, 1, H)
    b2r = b2.reshape(E, 1, D)
    gbr = gate_b.reshape(1, E)
    gammar = gamma.reshape(1, D)
    betar = beta.reshape(1, D)

    out = pl.pallas_call(
        functools.partial(_moe_body, ne=E, nhc=nhc),
        grid=(E, nhc),
        in_specs=[
            pl.BlockSpec((T, D), lambda e, hc: (0, 0)),
            pl.BlockSpec((T, D), lambda e, hc: (0, 0)),
            pl.BlockSpec((D, E), lambda e, hc: (0, 0)),
            pl.BlockSpec((1, E), lambda e, hc: (0, 0)),
            pl.BlockSpec((1, D, HC), lambda e, hc: (e, 0, hc)),
            pl.BlockSpec((1, 1, HC), lambda e, hc: (e, 0, hc)),
            pl.BlockSpec((1, HC, D), lambda e, hc: (e, hc, 0)),
            pl.BlockSpec((1, 1, D), lambda e, hc: (e, 0, 0)),
            pl.BlockSpec((1, D), lambda e, hc: (0, 0)),
            pl.BlockSpec((1, D), lambda e, hc: (0, 0)),
        ],
        out_specs=pl.BlockSpec((T, D), lambda e, hc: (0, 0)),
        out_shape=jax.ShapeDtypeStruct((T, D), jnp.float32),
        scratch_shapes=[
            pltpu.VMEM((T, E), jnp.float32),
            pltpu.VMEM((T, D), jnp.float32),
        ],
    )(x2, xb, gate_w, gbr, w1b, b1r, w2b, b2r, gammar, betar)
    return out.reshape(B, T, D)
